# baseline (speedup 1.0000x reference)
import functools

import jax
import jax.numpy as jnp
from jax import lax
from jax.experimental import pallas as pl
from jax.experimental.pallas import tpu as pltpu
from jax.experimental.pallas import tpu_sc as plsc

B, A, CLS, H, W, T = 16, 5, 20, 19, 19, 8
S = H * W
C = A * CLS
BT = B * T
NC, NS, LANES = 2, 16, 16


def _sigmoid(x):
    return 1.0 / (1.0 + jnp.exp(-x))


def _bce(x, t):
    return jnp.maximum(x, 0.0) - x * t + jnp.log(1.0 + jnp.exp(-jnp.abs(x)))


def _softplus(x):
    return jnp.maximum(x, 0.0) + jnp.log(1.0 + jnp.exp(-jnp.abs(x)))


def _sc_gather_kernel(resp_hbm, idx_hbm, out_resp, tix_v, tiy_v, tib_v,
                      resp_v, sem):
    wid = lax.axis_index("s") * NC + lax.axis_index("c")

    @pl.when(wid < B)
    def _():
        b = wid
        base8 = pl.multiple_of(b * T, 8)
        pltpu.sync_copy(idx_hbm.at[pl.ds(base8, T)], tix_v)
        pltpu.sync_copy(idx_hbm.at[pl.ds(base8 + BT, T)], tiy_v)
        pltpu.sync_copy(idx_hbm.at[pl.ds(base8 + 2 * BT, T)], tib_v)

        lane = lax.iota(jnp.int32, LANES)
        t_r = jnp.minimum(lane, T - 1)
        ib = plsc.load_gather(tib_v, [t_r])
        ix = plsc.load_gather(tix_v, [t_r])
        iy = plsc.load_gather(tiy_v, [t_r])
        ridx = b * (A * S) + ib * S + iy * W + ix
        pltpu.async_copy(resp_hbm.at[ridx], resp_v, sem).wait()
        pltpu.sync_copy(resp_v.at[pl.ds(0, T)], out_resp.at[pl.ds(base8, T)])


def _sc_gather(resp_flat, idx_cat):
    mesh = plsc.VectorSubcoreMesh(core_axis_name="c", subcore_axis_name="s")
    fn = functools.partial(
        pl.kernel,
        out_type=jax.ShapeDtypeStruct((BT,), jnp.float32),
        mesh=mesh,
        scratch_types=[
            pltpu.VMEM((T,), jnp.int32),
            pltpu.VMEM((T,), jnp.int32),
            pltpu.VMEM((T,), jnp.int32),
            pltpu.VMEM((LANES,), jnp.float32),
            pltpu.SemaphoreType.DMA,
        ],
        compiler_params=pltpu.CompilerParams(needs_layout_passes=False),
    )(_sc_gather_kernel)
    return fn(resp_flat, idx_cat)


def _group_bcast_last(x, timod):
    xm = x * (timod == T - 1).astype(jnp.float32)
    out = xm
    for t in range(T - 1):
        out = out + jnp.roll(xm, t - (T - 1), axis=0) * (timod == t).astype(jnp.float32)
    return out


def _main_kernel(cls_ref, bb_ref, resp_ref, tbc_ref, idxc_ref, idx_s, out_ref,
                 cls_scr, bb_scr, bbm_scr, respm_scr, sem, sem_big):
    big_bb = pltpu.async_copy(bb_ref, bbm_scr, sem_big)
    big_resp = pltpu.async_copy(resp_ref, respm_scr, sem_big)
    copies = []
    for b in range(B):
        for t in range(T):
            k = b * T + t
            ix = idx_s[k]
            iy = idx_s[BT + k]
            copies.append(pltpu.async_copy(
                cls_ref.at[iy, ix, b], cls_scr.at[k], sem))
            copies.append(pltpu.async_copy(
                bb_ref.at[iy, ix, b], bb_scr.at[k], sem))

    idxc = idxc_ref[...]
    tixf = idxc[:, 0:1].astype(jnp.float32)
    tiyf = idxc[:, 1:2].astype(jnp.float32)
    tibc = idxc[:, 2:3]
    lblc = idxc[:, 3:4]
    tbc = tbc_ref[...]
    tbx, tby = tbc[:, 0:1], tbc[:, 1:2]
    tbw, tbh = tbc[:, 2:3], tbc[:, 3:4]

    sub_iota = lax.broadcasted_iota(jnp.int32, (BT, 1), 0)
    timod = sub_iota % T
    p = tibc * S + idxc[:, 1:2] * W + idxc[:, 0:1]

    dup_later = jnp.zeros((BT, 1), jnp.float32)
    dup_earlier = jnp.zeros((BT, 1), jnp.float32)
    for d in range(1, T):
        eq_back = (p == jnp.roll(p, d, axis=0)) & (timod >= d)
        dup_earlier += eq_back.astype(jnp.float32)
        eq_fwd = (p == jnp.roll(p, -d, axis=0)) & (timod <= T - 1 - d)
        dup_later += eq_fwd.astype(jnp.float32)
    valid_last = (dup_later < 0.5).astype(jnp.float32)
    valid_first = (dup_earlier < 0.5).astype(jnp.float32)

    big_bb.wait()
    big_resp.wait()
    bb4m = jnp.transpose(bbm_scr[...], (2, 3, 0, 1)).reshape(B, A, 4, H, W)
    resp_f = jnp.transpose(respm_scr[...], (2, 3, 0, 1))
    mox, moy = bb4m[:, :, 0], bb4m[:, :, 1]
    mow, moh = bb4m[:, :, 2], bb4m[:, :, 3]
    Xc = lax.broadcasted_iota(jnp.int32, (1, 1, 1, W), 3).astype(jnp.float32)
    Yc = lax.broadcasted_iota(jnp.int32, (1, 1, H, 1), 2).astype(jnp.float32)

    gx1 = tbx + tixf - tbw * 0.5
    gy1 = tby + tiyf - tbh * 0.5
    lgx1 = _group_bcast_last(gx1, timod)
    lgy1 = _group_bcast_last(gy1, timod)
    ltbw = _group_bcast_last(tbw, timod)
    ltbh = _group_bcast_last(tbh, timod)

    m7 = (timod == T - 1).astype(jnp.float32)

    def per_image(col):
        return jnp.sum((col * m7).reshape(B, T), axis=1)[:, None, None, None]

    Gx1 = per_image(gx1)
    Gy1 = per_image(gy1)
    GW = per_image(tbw)
    GH = per_image(tbh)

    Px1 = _sigmoid(mox) + Xc - mow * 0.5
    Py1 = _sigmoid(moy) + Yc - moh * 0.5
    DX = jnp.maximum(jnp.minimum(Px1 + mow, Gx1 + GW) - jnp.maximum(Px1, Gx1), 0.0)
    DY = jnp.maximum(jnp.minimum(Py1 + moh, Gy1 + GH) - jnp.maximum(Py1, Gy1), 0.0)
    INTER = DX * DY
    negm = INTER / (mow * moh + GW * GH - INTER) < 0.6
    neg_raw = jnp.sum(jnp.where(negm, _softplus(resp_f), 0.0))

    for cp in copies:
        cp.wait()
    bbrow = bb_scr[...]
    c20 = lax.broadcasted_iota(jnp.int32, (BT, 4 * A), 1)
    base4 = tibc * 4

    def comp(c):
        return jnp.sum(jnp.where(c20 == base4 + c, bbrow, 0.0),
                       axis=1, keepdims=True)

    ox, oy, ow, oh = comp(0), comp(1), comp(2), comp(3)

    px1 = _sigmoid(ox) + tixf - ow * 0.5
    py1 = _sigmoid(oy) + tiyf - oh * 0.5
    dx = jnp.maximum(jnp.minimum(px1 + ow, gx1 + tbw) - jnp.maximum(px1, gx1), 0.0)
    dy = jnp.maximum(jnp.minimum(py1 + oh, gy1 + tbh) - jnp.maximum(py1, gy1), 0.0)
    inter = dx * dy
    iou_t = inter / (ow * oh + tbw * tbh - inter)

    loss_xy = jnp.sum(_bce(ox, tbx) + _bce(oy, tby))
    loss_wh = jnp.sum((ow - tbw) ** 2 + (oh - tbh) ** 2)

    ldx = jnp.maximum(jnp.minimum(px1 + ow, lgx1 + ltbw) - jnp.maximum(px1, lgx1), 0.0)
    ldy = jnp.maximum(jnp.minimum(py1 + oh, lgy1 + ltbh) - jnp.maximum(py1, lgy1), 0.0)
    linter = ldx * ldy
    iou_last_t = linter / (ow * oh + ltbw * ltbh - linter)

    glog100 = cls_scr[...]
    base20 = tibc * CLS
    ciota = lax.broadcasted_iota(jnp.int32, (BT, C), 1)
    in_rng = (ciota >= base20) & (ciota < base20 + CLS)
    mx = jnp.max(jnp.where(in_rng, glog100, -jnp.inf), axis=1, keepdims=True)
    ex = jnp.where(in_rng, jnp.exp(glog100 - mx), 0.0)
    lse = mx + jnp.log(jnp.sum(ex, axis=1, keepdims=True))
    picked = jnp.sum(jnp.where(ciota == base20 + lblc, glog100, 0.0),
                     axis=1, keepdims=True)
    loss_cls = jnp.sum(lse - picked)

    scal = (jnp.where(sub_iota == 0, loss_xy, 0.0)
            + jnp.where(sub_iota == 1, loss_wh, 0.0)
            + jnp.where(sub_iota == 2, loss_cls, 0.0)
            + jnp.where(sub_iota == 3, neg_raw, 0.0))
    out_ref[...] = jnp.concatenate(
        [iou_t, iou_last_t, valid_last, valid_first, scal,
         jnp.zeros((BT, 3), jnp.float32)], axis=1)


def _epilogue_kernel(out1_ref, gresp_ref, out_ref):
    cols = out1_ref[...]
    resp = gresp_ref[...]
    iou_t = cols[:, 0:1]
    iou_last_t = cols[:, 1:2]
    valid_last = cols[:, 2:3]
    valid_first = cols[:, 3:4]
    scal = cols[:, 4:5]

    loss_pos = jnp.sum(valid_last * _bce(resp, iou_t))
    sub_neg = jnp.sum(valid_first * jnp.where(iou_last_t < 0.6,
                                              _softplus(resp), 0.0))
    sub_iota = lax.broadcasted_iota(jnp.int32, (BT, 1), 0)

    def pick(k):
        return jnp.sum(jnp.where(sub_iota == k, scal, 0.0))

    loss_xy, loss_wh, loss_cls, neg_raw = pick(0), pick(1), pick(2), pick(3)
    loss_neg = 0.5 * (neg_raw - sub_neg)

    inv_b = 1.0 / B
    i5 = lax.broadcasted_iota(jnp.int32, (5,), 0)
    out = (jnp.where(i5 == 0, loss_pos * inv_b, 0.0)
           + jnp.where(i5 == 1, loss_neg * inv_b, 0.0)
           + jnp.where(i5 == 2, loss_cls * inv_b, 0.0)
           + jnp.where(i5 == 3, loss_xy * inv_b, 0.0)
           + jnp.where(i5 == 4, loss_wh * inv_b * 5.0, 0.0))
    out_ref[...] = out


@jax.jit
def kernel(pred_cls, pred_response, pred_bboxes, tgt_box, tgt_label, tgt_ix,
           tgt_iy, tgt_ibox):
    tix = tgt_ix.astype(jnp.int32)
    tiy = tgt_iy.astype(jnp.int32)
    tib = tgt_ibox.astype(jnp.int32)
    lbl = tgt_label.astype(jnp.int32)

    idx_cat = jnp.concatenate(
        [tix.reshape(BT), tiy.reshape(BT), tib.reshape(BT)])
    g_resp = _sc_gather(pred_response.reshape(B * A * S), idx_cat)

    idxc = jnp.stack([tix.reshape(BT), tiy.reshape(BT), tib.reshape(BT),
                      lbl.reshape(BT)], axis=1)
    smem = pl.BlockSpec(memory_space=pltpu.SMEM)
    hbm = pl.BlockSpec(memory_space=pltpu.HBM)
    out1 = pl.pallas_call(
        _main_kernel,
        out_shape=jax.ShapeDtypeStruct((BT, 8), jnp.float32),
        in_specs=[hbm, hbm, hbm,
                  pl.BlockSpec((BT, 4), lambda: (0, 0)),
                  pl.BlockSpec((BT, 4), lambda: (0, 0)),
                  smem],
        scratch_shapes=[pltpu.VMEM((BT, C), jnp.float32),
                        pltpu.VMEM((BT, 4 * A), jnp.float32),
                        pltpu.VMEM((H, W, B, 4 * A), jnp.float32),
                        pltpu.VMEM((H, W, B, A), jnp.float32),
                        pltpu.SemaphoreType.DMA,
                        pltpu.SemaphoreType.DMA],
    )(jnp.transpose(pred_cls, (2, 3, 0, 1)),
      jnp.transpose(pred_bboxes, (2, 3, 0, 1)),
      jnp.transpose(pred_response, (2, 3, 0, 1)),
      tgt_box.reshape(BT, 4), idxc, idx_cat)

    out = pl.pallas_call(
        _epilogue_kernel,
        out_shape=jax.ShapeDtypeStruct((5,), jnp.float32),
    )(out1, g_resp.reshape(BT, 1))
    return out


# speedup vs baseline: 1.1902x; 1.1902x over previous
import functools

import jax
import jax.numpy as jnp
from jax import lax
from jax.experimental import pallas as pl
from jax.experimental.pallas import tpu as pltpu
from jax.experimental.pallas import tpu_sc as plsc

B, A, CLS, H, W, T = 16, 5, 20, 19, 19, 8
S = H * W
C = A * CLS
BT = B * T
NC, NS, LANES = 2, 16, 16


def _sigmoid(x):
    return 1.0 / (1.0 + jnp.exp(-x))


def _bce(x, t):
    return jnp.maximum(x, 0.0) - x * t + jnp.log(1.0 + jnp.exp(-jnp.abs(x)))


def _softplus(x):
    return jnp.maximum(x, 0.0) + jnp.log(1.0 + jnp.exp(-jnp.abs(x)))


def _sc_gather_kernel(resp_hbm, idx_hbm, out_resp, tix_v, tiy_v, tib_v,
                      resp_v, sem):
    wid = lax.axis_index("s") * NC + lax.axis_index("c")

    @pl.when(wid < B)
    def _():
        b = wid
        base8 = pl.multiple_of(b * T, 8)
        pltpu.sync_copy(idx_hbm.at[pl.ds(base8, T)], tix_v)
        pltpu.sync_copy(idx_hbm.at[pl.ds(base8 + BT, T)], tiy_v)
        pltpu.sync_copy(idx_hbm.at[pl.ds(base8 + 2 * BT, T)], tib_v)

        lane = lax.iota(jnp.int32, LANES)
        t_r = jnp.minimum(lane, T - 1)
        ib = plsc.load_gather(tib_v, [t_r])
        ix = plsc.load_gather(tix_v, [t_r])
        iy = plsc.load_gather(tiy_v, [t_r])
        ridx = b * (A * S) + ib * S + iy * W + ix
        pltpu.async_copy(resp_hbm.at[ridx], resp_v, sem).wait()
        pltpu.sync_copy(resp_v.at[pl.ds(0, T)], out_resp.at[pl.ds(base8, T)])


def _sc_gather(resp_flat, idx_cat):
    mesh = plsc.VectorSubcoreMesh(core_axis_name="c", subcore_axis_name="s")
    fn = functools.partial(
        pl.kernel,
        out_type=jax.ShapeDtypeStruct((BT,), jnp.float32),
        mesh=mesh,
        scratch_types=[
            pltpu.VMEM((T,), jnp.int32),
            pltpu.VMEM((T,), jnp.int32),
            pltpu.VMEM((T,), jnp.int32),
            pltpu.VMEM((LANES,), jnp.float32),
            pltpu.SemaphoreType.DMA,
        ],
        compiler_params=pltpu.CompilerParams(needs_layout_passes=False),
    )(_sc_gather_kernel)
    return fn(resp_flat, idx_cat)


def _group_bcast_last(x, timod):
    xm = x * (timod == T - 1).astype(jnp.float32)
    out = xm
    for t in range(T - 1):
        out = out + jnp.roll(xm, t - (T - 1), axis=0) * (timod == t).astype(jnp.float32)
    return out


def _main_kernel(cls_ref, bb_ref, resp_ref, tbc_ref, idxc_ref, idx_s, out_ref,
                 cls_scr, bb_scr, bbm_scr, sem, sem_big):
    big_bb = pltpu.async_copy(bb_ref, bbm_scr, sem_big)
    copies = []
    for b in range(B):
        for t in range(T):
            k = b * T + t
            ix = idx_s[k]
            iy = idx_s[BT + k]
            copies.append(pltpu.async_copy(
                cls_ref.at[iy, ix, b], cls_scr.at[k], sem))
            copies.append(pltpu.async_copy(
                bb_ref.at[iy, ix, b], bb_scr.at[k], sem))

    idxc = idxc_ref[...]
    tixf = idxc[:, 0:1].astype(jnp.float32)
    tiyf = idxc[:, 1:2].astype(jnp.float32)
    tibc = idxc[:, 2:3]
    lblc = idxc[:, 3:4]
    tbc = tbc_ref[...]
    tbx, tby = tbc[:, 0:1], tbc[:, 1:2]
    tbw, tbh = tbc[:, 2:3], tbc[:, 3:4]

    sub_iota = lax.broadcasted_iota(jnp.int32, (BT, 1), 0)
    timod = sub_iota % T
    p = tibc * S + idxc[:, 1:2] * W + idxc[:, 0:1]

    dup_later = jnp.zeros((BT, 1), jnp.float32)
    dup_earlier = jnp.zeros((BT, 1), jnp.float32)
    for d in range(1, T):
        eq_back = (p == jnp.roll(p, d, axis=0)) & (timod >= d)
        dup_earlier += eq_back.astype(jnp.float32)
        eq_fwd = (p == jnp.roll(p, -d, axis=0)) & (timod <= T - 1 - d)
        dup_later += eq_fwd.astype(jnp.float32)
    valid_last = (dup_later < 0.5).astype(jnp.float32)
    valid_first = (dup_earlier < 0.5).astype(jnp.float32)

    big_bb.wait()
    bb4m = jnp.transpose(bbm_scr[...], (2, 3, 0, 1)).reshape(B, A, 4, H, W)
    resp_f = resp_ref[...]
    mox, moy = bb4m[:, :, 0], bb4m[:, :, 1]
    mow, moh = bb4m[:, :, 2], bb4m[:, :, 3]
    Xc = lax.broadcasted_iota(jnp.int32, (1, 1, 1, W), 3).astype(jnp.float32)
    Yc = lax.broadcasted_iota(jnp.int32, (1, 1, H, 1), 2).astype(jnp.float32)

    gx1 = tbx + tixf - tbw * 0.5
    gy1 = tby + tiyf - tbh * 0.5
    lgx1 = _group_bcast_last(gx1, timod)
    lgy1 = _group_bcast_last(gy1, timod)
    ltbw = _group_bcast_last(tbw, timod)
    ltbh = _group_bcast_last(tbh, timod)

    m7 = (timod == T - 1).astype(jnp.float32)

    def per_image(col):
        return jnp.sum((col * m7).reshape(B, T), axis=1)[:, None, None, None]

    Gx1 = per_image(gx1)
    Gy1 = per_image(gy1)
    GW = per_image(tbw)
    GH = per_image(tbh)

    Px1 = _sigmoid(mox) + Xc - mow * 0.5
    Py1 = _sigmoid(moy) + Yc - moh * 0.5
    DX = jnp.maximum(jnp.minimum(Px1 + mow, Gx1 + GW) - jnp.maximum(Px1, Gx1), 0.0)
    DY = jnp.maximum(jnp.minimum(Py1 + moh, Gy1 + GH) - jnp.maximum(Py1, Gy1), 0.0)
    INTER = DX * DY
    negm = INTER / (mow * moh + GW * GH - INTER) < 0.6
    neg_raw = jnp.sum(jnp.where(negm, _softplus(resp_f), 0.0))

    for cp in copies:
        cp.wait()
    bbrow = bb_scr[...]
    c20 = lax.broadcasted_iota(jnp.int32, (BT, 4 * A), 1)
    base4 = tibc * 4

    def comp(c):
        return jnp.sum(jnp.where(c20 == base4 + c, bbrow, 0.0),
                       axis=1, keepdims=True)

    ox, oy, ow, oh = comp(0), comp(1), comp(2), comp(3)

    px1 = _sigmoid(ox) + tixf - ow * 0.5
    py1 = _sigmoid(oy) + tiyf - oh * 0.5
    dx = jnp.maximum(jnp.minimum(px1 + ow, gx1 + tbw) - jnp.maximum(px1, gx1), 0.0)
    dy = jnp.maximum(jnp.minimum(py1 + oh, gy1 + tbh) - jnp.maximum(py1, gy1), 0.0)
    inter = dx * dy
    iou_t = inter / (ow * oh + tbw * tbh - inter)

    loss_xy = jnp.sum(_bce(ox, tbx) + _bce(oy, tby))
    loss_wh = jnp.sum((ow - tbw) ** 2 + (oh - tbh) ** 2)

    ldx = jnp.maximum(jnp.minimum(px1 + ow, lgx1 + ltbw) - jnp.maximum(px1, lgx1), 0.0)
    ldy = jnp.maximum(jnp.minimum(py1 + oh, lgy1 + ltbh) - jnp.maximum(py1, lgy1), 0.0)
    linter = ldx * ldy
    iou_last_t = linter / (ow * oh + ltbw * ltbh - linter)

    glog100 = cls_scr[...]
    base20 = tibc * CLS
    ciota = lax.broadcasted_iota(jnp.int32, (BT, C), 1)
    in_rng = (ciota >= base20) & (ciota < base20 + CLS)
    mx = jnp.max(jnp.where(in_rng, glog100, -jnp.inf), axis=1, keepdims=True)
    ex = jnp.where(in_rng, jnp.exp(glog100 - mx), 0.0)
    lse = mx + jnp.log(jnp.sum(ex, axis=1, keepdims=True))
    picked = jnp.sum(jnp.where(ciota == base20 + lblc, glog100, 0.0),
                     axis=1, keepdims=True)
    loss_cls = jnp.sum(lse - picked)

    lanev = lax.broadcasted_iota(jnp.int32, (1, BT), 1)
    scal = (jnp.where(lanev == 0, loss_xy, 0.0)
            + jnp.where(lanev == 1, loss_wh, 0.0)
            + jnp.where(lanev == 2, loss_cls, 0.0)
            + jnp.where(lanev == 3, neg_raw, 0.0))
    out_ref[...] = jnp.concatenate(
        [jnp.transpose(iou_t), jnp.transpose(iou_last_t),
         jnp.transpose(valid_last), jnp.transpose(valid_first), scal,
         jnp.zeros((3, BT), jnp.float32)], axis=0)


def _epilogue_kernel(out1_ref, gresp_ref, out_ref):
    rows = out1_ref[...]
    resp = gresp_ref[...]
    iou_t = rows[0:1, :]
    iou_last_t = rows[1:2, :]
    valid_last = rows[2:3, :]
    valid_first = rows[3:4, :]
    scal = rows[4:5, :]

    loss_pos = jnp.sum(valid_last * _bce(resp, iou_t))
    sub_neg = jnp.sum(valid_first * jnp.where(iou_last_t < 0.6,
                                              _softplus(resp), 0.0))
    lanev = lax.broadcasted_iota(jnp.int32, (1, BT), 1)

    def pick(k):
        return jnp.sum(jnp.where(lanev == k, scal, 0.0))

    loss_xy, loss_wh, loss_cls, neg_raw = pick(0), pick(1), pick(2), pick(3)
    loss_neg = 0.5 * (neg_raw - sub_neg)

    inv_b = 1.0 / B
    i5 = lax.broadcasted_iota(jnp.int32, (5,), 0)
    out = (jnp.where(i5 == 0, loss_pos * inv_b, 0.0)
           + jnp.where(i5 == 1, loss_neg * inv_b, 0.0)
           + jnp.where(i5 == 2, loss_cls * inv_b, 0.0)
           + jnp.where(i5 == 3, loss_xy * inv_b, 0.0)
           + jnp.where(i5 == 4, loss_wh * inv_b * 5.0, 0.0))
    out_ref[...] = out


@jax.jit
def kernel(pred_cls, pred_response, pred_bboxes, tgt_box, tgt_label, tgt_ix,
           tgt_iy, tgt_ibox):
    tix = tgt_ix.astype(jnp.int32)
    tiy = tgt_iy.astype(jnp.int32)
    tib = tgt_ibox.astype(jnp.int32)
    lbl = tgt_label.astype(jnp.int32)

    idx_cat = jnp.concatenate(
        [tix.reshape(BT), tiy.reshape(BT), tib.reshape(BT)])
    g_resp = _sc_gather(pred_response.reshape(B * A * S), idx_cat)

    idxc = jnp.stack([tix.reshape(BT), tiy.reshape(BT), tib.reshape(BT),
                      lbl.reshape(BT)], axis=1)
    smem = pl.BlockSpec(memory_space=pltpu.SMEM)
    hbm = pl.BlockSpec(memory_space=pltpu.HBM)
    out1 = pl.pallas_call(
        _main_kernel,
        out_shape=jax.ShapeDtypeStruct((8, BT), jnp.float32),
        in_specs=[hbm, hbm,
                  pl.BlockSpec((B, A, H, W), lambda: (0, 0, 0, 0)),
                  pl.BlockSpec((BT, 4), lambda: (0, 0)),
                  pl.BlockSpec((BT, 4), lambda: (0, 0)),
                  smem],
        scratch_shapes=[pltpu.VMEM((BT, C), jnp.float32),
                        pltpu.VMEM((BT, 4 * A), jnp.float32),
                        pltpu.VMEM((H, W, B, 4 * A), jnp.float32),
                        pltpu.SemaphoreType.DMA,
                        pltpu.SemaphoreType.DMA],
    )(jnp.transpose(pred_cls, (2, 3, 0, 1)),
      jnp.transpose(pred_bboxes, (2, 3, 0, 1)),
      pred_response,
      tgt_box.reshape(BT, 4), idxc, idx_cat)

    out = pl.pallas_call(
        _epilogue_kernel,
        out_shape=jax.ShapeDtypeStruct((5,), jnp.float32),
    )(out1, g_resp.reshape(1, BT))
    return out


# speedup vs baseline: 1.2303x; 1.0337x over previous
import functools

import jax
import jax.numpy as jnp
from jax import lax
from jax.experimental import pallas as pl
from jax.experimental.pallas import tpu as pltpu
from jax.experimental.pallas import tpu_sc as plsc

B, A, CLS, H, W, T = 16, 5, 20, 19, 19, 8
S = H * W
C = A * CLS
BT = B * T
NC, NS, LANES = 2, 16, 16


def _sigmoid(x):
    return 1.0 / (1.0 + jnp.exp(-x))


def _bce(x, t):
    return jnp.maximum(x, 0.0) - x * t + jnp.log(1.0 + jnp.exp(-jnp.abs(x)))


def _softplus(x):
    return jnp.maximum(x, 0.0) + jnp.log(1.0 + jnp.exp(-jnp.abs(x)))


def _sc_gather_kernel(resp_hbm, idx_hbm, out_resp, tix_v, tiy_v, tib_v,
                      resp_v, sem):
    wid = lax.axis_index("s") * NC + lax.axis_index("c")

    @pl.when(wid < B)
    def _():
        b = wid
        base8 = pl.multiple_of(b * T, 8)
        pltpu.sync_copy(idx_hbm.at[pl.ds(base8, T)], tix_v)
        pltpu.sync_copy(idx_hbm.at[pl.ds(base8 + BT, T)], tiy_v)
        pltpu.sync_copy(idx_hbm.at[pl.ds(base8 + 2 * BT, T)], tib_v)

        lane = lax.iota(jnp.int32, LANES)
        t_r = jnp.minimum(lane, T - 1)
        ib = plsc.load_gather(tib_v, [t_r])
        ix = plsc.load_gather(tix_v, [t_r])
        iy = plsc.load_gather(tiy_v, [t_r])
        ridx = b * (A * S) + ib * S + iy * W + ix
        pltpu.async_copy(resp_hbm.at[ridx], resp_v, sem).wait()
        pltpu.sync_copy(resp_v.at[pl.ds(0, T)], out_resp.at[pl.ds(base8, T)])


def _sc_gather(resp_flat, idx_cat):
    mesh = plsc.VectorSubcoreMesh(core_axis_name="c", subcore_axis_name="s")
    fn = functools.partial(
        pl.kernel,
        out_type=jax.ShapeDtypeStruct((BT,), jnp.float32),
        mesh=mesh,
        scratch_types=[
            pltpu.VMEM((T,), jnp.int32),
            pltpu.VMEM((T,), jnp.int32),
            pltpu.VMEM((T,), jnp.int32),
            pltpu.VMEM((LANES,), jnp.float32),
            pltpu.SemaphoreType.DMA,
        ],
        compiler_params=pltpu.CompilerParams(needs_layout_passes=False),
    )(_sc_gather_kernel)
    return fn(resp_flat, idx_cat)


def _group_bcast_last(x, timod):
    xm = x * (timod == T - 1).astype(jnp.float32)
    out = xm
    for t in range(T - 1):
        out = out + jnp.roll(xm, t - (T - 1), axis=0) * (timod == t).astype(jnp.float32)
    return out


def _main_kernel(cls_ref, bb_ref, resp_ref, tbc_ref, idxc_ref, idx_s, out_ref,
                 cls_scr, bb_scr, bbm_scr, sem, sem_big):
    copies = []
    for b in range(B):
        for t in range(T):
            k = b * T + t
            ix = idx_s[k]
            iy = idx_s[BT + k]
            copies.append(pltpu.async_copy(
                cls_ref.at[iy, ix, b], cls_scr.at[k], sem))
            copies.append(pltpu.async_copy(
                bb_ref.at[iy, ix, b], bb_scr.at[k], sem))
    big_bb = pltpu.async_copy(bb_ref, bbm_scr, sem_big)

    idxc = idxc_ref[...]
    tixf = idxc[:, 0:1].astype(jnp.float32)
    tiyf = idxc[:, 1:2].astype(jnp.float32)
    tibc = idxc[:, 2:3]
    lblc = idxc[:, 3:4]
    tbc = tbc_ref[...]
    tbx, tby = tbc[:, 0:1], tbc[:, 1:2]
    tbw, tbh = tbc[:, 2:3], tbc[:, 3:4]

    sub_iota = lax.broadcasted_iota(jnp.int32, (BT, 1), 0)
    timod = sub_iota % T
    p = tibc * S + idxc[:, 1:2] * W + idxc[:, 0:1]

    dup_later = jnp.zeros((BT, 1), jnp.float32)
    dup_earlier = jnp.zeros((BT, 1), jnp.float32)
    for d in range(1, T):
        eq_back = (p == jnp.roll(p, d, axis=0)) & (timod >= d)
        dup_earlier += eq_back.astype(jnp.float32)
        eq_fwd = (p == jnp.roll(p, -d, axis=0)) & (timod <= T - 1 - d)
        dup_later += eq_fwd.astype(jnp.float32)
    valid_last = (dup_later < 0.5).astype(jnp.float32)
    valid_first = (dup_earlier < 0.5).astype(jnp.float32)

    gx1 = tbx + tixf - tbw * 0.5
    gy1 = tby + tiyf - tbh * 0.5
    lgx1 = _group_bcast_last(gx1, timod)
    lgy1 = _group_bcast_last(gy1, timod)
    ltbw = _group_bcast_last(tbw, timod)
    ltbh = _group_bcast_last(tbh, timod)

    for cp in copies:
        cp.wait()
    bbrow = bb_scr[...]
    c20 = lax.broadcasted_iota(jnp.int32, (BT, 4 * A), 1)
    base4 = tibc * 4

    def comp(c):
        return jnp.sum(jnp.where(c20 == base4 + c, bbrow, 0.0),
                       axis=1, keepdims=True)

    ox, oy, ow, oh = comp(0), comp(1), comp(2), comp(3)

    px1 = _sigmoid(ox) + tixf - ow * 0.5
    py1 = _sigmoid(oy) + tiyf - oh * 0.5
    dx = jnp.maximum(jnp.minimum(px1 + ow, gx1 + tbw) - jnp.maximum(px1, gx1), 0.0)
    dy = jnp.maximum(jnp.minimum(py1 + oh, gy1 + tbh) - jnp.maximum(py1, gy1), 0.0)
    inter = dx * dy
    iou_t = inter / (ow * oh + tbw * tbh - inter)

    loss_xy = jnp.sum(_bce(ox, tbx) + _bce(oy, tby))
    loss_wh = jnp.sum((ow - tbw) ** 2 + (oh - tbh) ** 2)

    ldx = jnp.maximum(jnp.minimum(px1 + ow, lgx1 + ltbw) - jnp.maximum(px1, lgx1), 0.0)
    ldy = jnp.maximum(jnp.minimum(py1 + oh, lgy1 + ltbh) - jnp.maximum(py1, lgy1), 0.0)
    linter = ldx * ldy
    iou_last_t = linter / (ow * oh + ltbw * ltbh - linter)

    glog100 = cls_scr[...]
    base20 = tibc * CLS
    ciota = lax.broadcasted_iota(jnp.int32, (BT, C), 1)
    in_rng = (ciota >= base20) & (ciota < base20 + CLS)
    mx = jnp.max(jnp.where(in_rng, glog100, -jnp.inf), axis=1, keepdims=True)
    ex = jnp.where(in_rng, jnp.exp(glog100 - mx), 0.0)
    lse = mx + jnp.log(jnp.sum(ex, axis=1, keepdims=True))
    picked = jnp.sum(jnp.where(ciota == base20 + lblc, glog100, 0.0),
                     axis=1, keepdims=True)
    loss_cls = jnp.sum(lse - picked)

    big_bb.wait()
    bb4m = jnp.transpose(bbm_scr[...], (2, 3, 0, 1)).reshape(B, A, 4, H, W)
    resp_f = resp_ref[...]
    mox, moy = bb4m[:, :, 0], bb4m[:, :, 1]
    mow, moh = bb4m[:, :, 2], bb4m[:, :, 3]
    Xc = lax.broadcasted_iota(jnp.int32, (1, 1, 1, W), 3).astype(jnp.float32)
    Yc = lax.broadcasted_iota(jnp.int32, (1, 1, H, 1), 2).astype(jnp.float32)

    m7 = (timod == T - 1).astype(jnp.float32)

    def per_image(col):
        return jnp.sum((col * m7).reshape(B, T), axis=1)[:, None, None, None]

    Gx1 = per_image(gx1)
    Gy1 = per_image(gy1)
    GW = per_image(tbw)
    GH = per_image(tbh)

    Px1 = _sigmoid(mox) + Xc - mow * 0.5
    Py1 = _sigmoid(moy) + Yc - moh * 0.5
    DX = jnp.maximum(jnp.minimum(Px1 + mow, Gx1 + GW) - jnp.maximum(Px1, Gx1), 0.0)
    DY = jnp.maximum(jnp.minimum(Py1 + moh, Gy1 + GH) - jnp.maximum(Py1, Gy1), 0.0)
    INTER = DX * DY
    negm = INTER / (mow * moh + GW * GH - INTER) < 0.6
    neg_raw = jnp.sum(jnp.where(negm, _softplus(resp_f), 0.0))

    lanev = lax.broadcasted_iota(jnp.int32, (1, BT), 1)
    scal = (jnp.where(lanev == 0, loss_xy, 0.0)
            + jnp.where(lanev == 1, loss_wh, 0.0)
            + jnp.where(lanev == 2, loss_cls, 0.0)
            + jnp.where(lanev == 3, neg_raw, 0.0))
    out_ref[...] = jnp.concatenate(
        [jnp.transpose(iou_t), jnp.transpose(iou_last_t),
         jnp.transpose(valid_last), jnp.transpose(valid_first), scal,
         jnp.zeros((3, BT), jnp.float32)], axis=0)


def _epilogue_kernel(out1_ref, gresp_ref, out_ref):
    rows = out1_ref[...]
    resp = gresp_ref[...]
    iou_t = rows[0:1, :]
    iou_last_t = rows[1:2, :]
    valid_last = rows[2:3, :]
    valid_first = rows[3:4, :]
    scal = rows[4:5, :]

    loss_pos = jnp.sum(valid_last * _bce(resp, iou_t))
    sub_neg = jnp.sum(valid_first * jnp.where(iou_last_t < 0.6,
                                              _softplus(resp), 0.0))
    lanev = lax.broadcasted_iota(jnp.int32, (1, BT), 1)

    def pick(k):
        return jnp.sum(jnp.where(lanev == k, scal, 0.0))

    loss_xy, loss_wh, loss_cls, neg_raw = pick(0), pick(1), pick(2), pick(3)
    loss_neg = 0.5 * (neg_raw - sub_neg)

    inv_b = 1.0 / B
    i5 = lax.broadcasted_iota(jnp.int32, (5,), 0)
    out = (jnp.where(i5 == 0, loss_pos * inv_b, 0.0)
           + jnp.where(i5 == 1, loss_neg * inv_b, 0.0)
           + jnp.where(i5 == 2, loss_cls * inv_b, 0.0)
           + jnp.where(i5 == 3, loss_xy * inv_b, 0.0)
           + jnp.where(i5 == 4, loss_wh * inv_b * 5.0, 0.0))
    out_ref[...] = out


@jax.jit
def kernel(pred_cls, pred_response, pred_bboxes, tgt_box, tgt_label, tgt_ix,
           tgt_iy, tgt_ibox):
    tix = tgt_ix.astype(jnp.int32)
    tiy = tgt_iy.astype(jnp.int32)
    tib = tgt_ibox.astype(jnp.int32)
    lbl = tgt_label.astype(jnp.int32)

    idx_cat = jnp.concatenate(
        [tix.reshape(BT), tiy.reshape(BT), tib.reshape(BT)])
    g_resp = _sc_gather(pred_response.reshape(B * A * S), idx_cat)

    idxc = jnp.stack([tix.reshape(BT), tiy.reshape(BT), tib.reshape(BT),
                      lbl.reshape(BT)], axis=1)
    smem = pl.BlockSpec(memory_space=pltpu.SMEM)
    hbm = pl.BlockSpec(memory_space=pltpu.HBM)
    out1 = pl.pallas_call(
        _main_kernel,
        out_shape=jax.ShapeDtypeStruct((8, BT), jnp.float32),
        in_specs=[hbm, hbm,
                  pl.BlockSpec((B, A, H, W), lambda: (0, 0, 0, 0)),
                  pl.BlockSpec((BT, 4), lambda: (0, 0)),
                  pl.BlockSpec((BT, 4), lambda: (0, 0)),
                  smem],
        scratch_shapes=[pltpu.VMEM((BT, C), jnp.float32),
                        pltpu.VMEM((BT, 4 * A), jnp.float32),
                        pltpu.VMEM((H, W, B, 4 * A), jnp.float32),
                        pltpu.SemaphoreType.DMA,
                        pltpu.SemaphoreType.DMA],
    )(jnp.transpose(pred_cls, (2, 3, 0, 1)),
      jnp.transpose(pred_bboxes, (2, 3, 0, 1)),
      pred_response,
      tgt_box.reshape(BT, 4), idxc, idx_cat)

    out = pl.pallas_call(
        _epilogue_kernel,
        out_shape=jax.ShapeDtypeStruct((5,), jnp.float32),
    )(out1, g_resp.reshape(1, BT))
    return out


# speedup vs baseline: 1.2351x; 1.0039x over previous
import functools

import jax
import jax.numpy as jnp
from jax import lax
from jax.experimental import pallas as pl
from jax.experimental.pallas import tpu as pltpu
from jax.experimental.pallas import tpu_sc as plsc

B, A, CLS, H, W, T = 16, 5, 20, 19, 19, 8
S = H * W
C = A * CLS
BT = B * T
NC, NS, LANES = 2, 16, 16


def _sigmoid(x):
    return 1.0 / (1.0 + jnp.exp(-x))


def _bce(x, t):
    return jnp.maximum(x, 0.0) - x * t + jnp.log(1.0 + jnp.exp(-jnp.abs(x)))


def _softplus(x):
    return jnp.maximum(x, 0.0) + jnp.log(1.0 + jnp.exp(-jnp.abs(x)))


def _sc_gather_kernel(resp_hbm, idx_hbm, out_resp, tix_v, tiy_v, tib_v,
                      resp_v, sem):
    wid = lax.axis_index("s") * NC + lax.axis_index("c")

    @pl.when(wid < B)
    def _():
        b = wid
        base8 = pl.multiple_of(b * T, 8)
        pltpu.sync_copy(idx_hbm.at[pl.ds(base8, T)], tix_v)
        pltpu.sync_copy(idx_hbm.at[pl.ds(base8 + BT, T)], tiy_v)
        pltpu.sync_copy(idx_hbm.at[pl.ds(base8 + 2 * BT, T)], tib_v)

        lane = lax.iota(jnp.int32, LANES)
        t_r = jnp.minimum(lane, T - 1)
        ib = plsc.load_gather(tib_v, [t_r])
        ix = plsc.load_gather(tix_v, [t_r])
        iy = plsc.load_gather(tiy_v, [t_r])
        ridx = b * (A * S) + ib * S + iy * W + ix
        pltpu.async_copy(resp_hbm.at[ridx], resp_v, sem).wait()
        pltpu.sync_copy(resp_v.at[pl.ds(0, T)], out_resp.at[pl.ds(base8, T)])


def _sc_gather(resp_flat, idx_cat):
    mesh = plsc.VectorSubcoreMesh(core_axis_name="c", subcore_axis_name="s")
    fn = functools.partial(
        pl.kernel,
        out_type=jax.ShapeDtypeStruct((BT,), jnp.float32),
        mesh=mesh,
        scratch_types=[
            pltpu.VMEM((T,), jnp.int32),
            pltpu.VMEM((T,), jnp.int32),
            pltpu.VMEM((T,), jnp.int32),
            pltpu.VMEM((LANES,), jnp.float32),
            pltpu.SemaphoreType.DMA,
        ],
        compiler_params=pltpu.CompilerParams(needs_layout_passes=False),
    )(_sc_gather_kernel)
    return fn(resp_flat, idx_cat)


def _group_bcast_last(x, timod):
    xm = x * (timod == T - 1).astype(jnp.float32)
    out = xm
    for t in range(T - 1):
        out = out + jnp.roll(xm, t - (T - 1), axis=0) * (timod == t).astype(jnp.float32)
    return out


def _main_kernel(cls_ref, bb_ref, resp_ref, tbc_ref, idxc_ref, idx_s, out_ref,
                 cls_scr, bb_scr, bbm_scr, sem, sem_bb, sem_big):
    cls_copies = []
    bb_copies = []
    for b in range(B):
        for t in range(T):
            k = b * T + t
            ix = idx_s[k]
            iy = idx_s[BT + k]
            bb_copies.append(pltpu.async_copy(
                bb_ref.at[iy, ix, b], bb_scr.at[k], sem_bb))
            cls_copies.append(pltpu.async_copy(
                cls_ref.at[iy, ix, b], cls_scr.at[k], sem))
    big_bb = pltpu.async_copy(bb_ref, bbm_scr, sem_big)

    idxc = idxc_ref[...]
    tixf = idxc[:, 0:1].astype(jnp.float32)
    tiyf = idxc[:, 1:2].astype(jnp.float32)
    tibc = idxc[:, 2:3]
    lblc = idxc[:, 3:4]
    tbc = tbc_ref[...]
    tbx, tby = tbc[:, 0:1], tbc[:, 1:2]
    tbw, tbh = tbc[:, 2:3], tbc[:, 3:4]

    sub_iota = lax.broadcasted_iota(jnp.int32, (BT, 1), 0)
    timod = sub_iota % T
    p = tibc * S + idxc[:, 1:2] * W + idxc[:, 0:1]

    dup_later = jnp.zeros((BT, 1), jnp.float32)
    dup_earlier = jnp.zeros((BT, 1), jnp.float32)
    for d in range(1, T):
        eq_back = (p == jnp.roll(p, d, axis=0)) & (timod >= d)
        dup_earlier += eq_back.astype(jnp.float32)
        eq_fwd = (p == jnp.roll(p, -d, axis=0)) & (timod <= T - 1 - d)
        dup_later += eq_fwd.astype(jnp.float32)
    valid_last = (dup_later < 0.5).astype(jnp.float32)
    valid_first = (dup_earlier < 0.5).astype(jnp.float32)

    gx1 = tbx + tixf - tbw * 0.5
    gy1 = tby + tiyf - tbh * 0.5
    lgx1 = _group_bcast_last(gx1, timod)
    lgy1 = _group_bcast_last(gy1, timod)
    ltbw = _group_bcast_last(tbw, timod)
    ltbh = _group_bcast_last(tbh, timod)

    for cp in bb_copies:
        cp.wait()
    bbrow = bb_scr[...]
    c20 = lax.broadcasted_iota(jnp.int32, (BT, 4 * A), 1)
    base4 = tibc * 4

    def comp(c):
        return jnp.sum(jnp.where(c20 == base4 + c, bbrow, 0.0),
                       axis=1, keepdims=True)

    ox, oy, ow, oh = comp(0), comp(1), comp(2), comp(3)

    px1 = _sigmoid(ox) + tixf - ow * 0.5
    py1 = _sigmoid(oy) + tiyf - oh * 0.5
    dx = jnp.maximum(jnp.minimum(px1 + ow, gx1 + tbw) - jnp.maximum(px1, gx1), 0.0)
    dy = jnp.maximum(jnp.minimum(py1 + oh, gy1 + tbh) - jnp.maximum(py1, gy1), 0.0)
    inter = dx * dy
    iou_t = inter / (ow * oh + tbw * tbh - inter)

    loss_xy = jnp.sum(_bce(ox, tbx) + _bce(oy, tby))
    loss_wh = jnp.sum((ow - tbw) ** 2 + (oh - tbh) ** 2)

    ldx = jnp.maximum(jnp.minimum(px1 + ow, lgx1 + ltbw) - jnp.maximum(px1, lgx1), 0.0)
    ldy = jnp.maximum(jnp.minimum(py1 + oh, lgy1 + ltbh) - jnp.maximum(py1, lgy1), 0.0)
    linter = ldx * ldy
    iou_last_t = linter / (ow * oh + ltbw * ltbh - linter)

    for cp in cls_copies:
        cp.wait()
    glog100 = cls_scr[...]
    base20 = tibc * CLS
    ciota = lax.broadcasted_iota(jnp.int32, (BT, C), 1)
    in_rng = (ciota >= base20) & (ciota < base20 + CLS)
    mx = jnp.max(jnp.where(in_rng, glog100, -jnp.inf), axis=1, keepdims=True)
    ex = jnp.where(in_rng, jnp.exp(glog100 - mx), 0.0)
    lse = mx + jnp.log(jnp.sum(ex, axis=1, keepdims=True))
    picked = jnp.sum(jnp.where(ciota == base20 + lblc, glog100, 0.0),
                     axis=1, keepdims=True)
    loss_cls = jnp.sum(lse - picked)

    big_bb.wait()
    bb4m = jnp.transpose(bbm_scr[...], (2, 3, 0, 1)).reshape(B, A, 4, H, W)
    resp_f = resp_ref[...]
    mox, moy = bb4m[:, :, 0], bb4m[:, :, 1]
    mow, moh = bb4m[:, :, 2], bb4m[:, :, 3]
    Xc = lax.broadcasted_iota(jnp.int32, (1, 1, 1, W), 3).astype(jnp.float32)
    Yc = lax.broadcasted_iota(jnp.int32, (1, 1, H, 1), 2).astype(jnp.float32)

    m7 = (timod == T - 1).astype(jnp.float32)

    def per_image(col):
        return jnp.sum((col * m7).reshape(B, T), axis=1)[:, None, None, None]

    Gx1 = per_image(gx1)
    Gy1 = per_image(gy1)
    GW = per_image(tbw)
    GH = per_image(tbh)

    Px1 = _sigmoid(mox) + Xc - mow * 0.5
    Py1 = _sigmoid(moy) + Yc - moh * 0.5
    DX = jnp.maximum(jnp.minimum(Px1 + mow, Gx1 + GW) - jnp.maximum(Px1, Gx1), 0.0)
    DY = jnp.maximum(jnp.minimum(Py1 + moh, Gy1 + GH) - jnp.maximum(Py1, Gy1), 0.0)
    INTER = DX * DY
    negm = INTER / (mow * moh + GW * GH - INTER) < 0.6
    neg_raw = jnp.sum(jnp.where(negm, _softplus(resp_f), 0.0))

    lanev = lax.broadcasted_iota(jnp.int32, (1, BT), 1)
    scal = (jnp.where(lanev == 0, loss_xy, 0.0)
            + jnp.where(lanev == 1, loss_wh, 0.0)
            + jnp.where(lanev == 2, loss_cls, 0.0)
            + jnp.where(lanev == 3, neg_raw, 0.0))
    out_ref[...] = jnp.concatenate(
        [jnp.transpose(iou_t), jnp.transpose(iou_last_t),
         jnp.transpose(valid_last), jnp.transpose(valid_first), scal,
         jnp.zeros((3, BT), jnp.float32)], axis=0)


def _epilogue_kernel(out1_ref, gresp_ref, out_ref):
    rows = out1_ref[...]
    resp = gresp_ref[...]
    iou_t = rows[0:1, :]
    iou_last_t = rows[1:2, :]
    valid_last = rows[2:3, :]
    valid_first = rows[3:4, :]
    scal = rows[4:5, :]

    loss_pos = jnp.sum(valid_last * _bce(resp, iou_t))
    sub_neg = jnp.sum(valid_first * jnp.where(iou_last_t < 0.6,
                                              _softplus(resp), 0.0))
    lanev = lax.broadcasted_iota(jnp.int32, (1, BT), 1)

    def pick(k):
        return jnp.sum(jnp.where(lanev == k, scal, 0.0))

    loss_xy, loss_wh, loss_cls, neg_raw = pick(0), pick(1), pick(2), pick(3)
    loss_neg = 0.5 * (neg_raw - sub_neg)

    inv_b = 1.0 / B
    i5 = lax.broadcasted_iota(jnp.int32, (5,), 0)
    out = (jnp.where(i5 == 0, loss_pos * inv_b, 0.0)
           + jnp.where(i5 == 1, loss_neg * inv_b, 0.0)
           + jnp.where(i5 == 2, loss_cls * inv_b, 0.0)
           + jnp.where(i5 == 3, loss_xy * inv_b, 0.0)
           + jnp.where(i5 == 4, loss_wh * inv_b * 5.0, 0.0))
    out_ref[...] = out


@jax.jit
def kernel(pred_cls, pred_response, pred_bboxes, tgt_box, tgt_label, tgt_ix,
           tgt_iy, tgt_ibox):
    tix = tgt_ix.astype(jnp.int32)
    tiy = tgt_iy.astype(jnp.int32)
    tib = tgt_ibox.astype(jnp.int32)
    lbl = tgt_label.astype(jnp.int32)

    idx_cat = jnp.concatenate(
        [tix.reshape(BT), tiy.reshape(BT), tib.reshape(BT)])
    g_resp = _sc_gather(pred_response.reshape(B * A * S), idx_cat)

    idxc = jnp.stack([tix.reshape(BT), tiy.reshape(BT), tib.reshape(BT),
                      lbl.reshape(BT)], axis=1)
    smem = pl.BlockSpec(memory_space=pltpu.SMEM)
    hbm = pl.BlockSpec(memory_space=pltpu.HBM)
    out1 = pl.pallas_call(
        _main_kernel,
        out_shape=jax.ShapeDtypeStruct((8, BT), jnp.float32),
        in_specs=[hbm, hbm,
                  pl.BlockSpec((B, A, H, W), lambda: (0, 0, 0, 0)),
                  pl.BlockSpec((BT, 4), lambda: (0, 0)),
                  pl.BlockSpec((BT, 4), lambda: (0, 0)),
                  smem],
        scratch_shapes=[pltpu.VMEM((BT, C), jnp.float32),
                        pltpu.VMEM((BT, 4 * A), jnp.float32),
                        pltpu.VMEM((H, W, B, 4 * A), jnp.float32),
                        pltpu.SemaphoreType.DMA,
                        pltpu.SemaphoreType.DMA,
                        pltpu.SemaphoreType.DMA],
    )(jnp.transpose(pred_cls, (2, 3, 0, 1)),
      jnp.transpose(pred_bboxes, (2, 3, 0, 1)),
      pred_response,
      tgt_box.reshape(BT, 4), idxc, idx_cat)

    out = pl.pallas_call(
        _epilogue_kernel,
        out_shape=jax.ShapeDtypeStruct((5,), jnp.float32),
    )(out1, g_resp.reshape(1, BT))
    return out
